# Initial kernel scaffold; baseline (speedup 1.0000x reference)
import jax
import jax.numpy as jnp
from jax.experimental import pallas as pl


def kernel(x, embed):
    raise NotImplementedError("write your pallas kernel here")


# speedup vs baseline: 2.8426x; 2.8426x over previous
import functools

import jax
import jax.numpy as jnp
from jax import lax
from jax.experimental import pallas as pl

B = 32
T = 576
DIM = 256
K = 1024
N = B * T
BN = 512
NB = N // BN


def _vq_body(x_ref, e_ref, dist_ref, ind_ref, q_ref):
    x = x_ref[...]
    e = e_ref[...]
    prod = lax.dot_general(x, e, (((1,), (1,)), ((), ())),
                           preferred_element_type=jnp.float32)
    zsq = jnp.sum(x * x, axis=1, keepdims=True)
    esq = jnp.sum(e * e, axis=1)[None, :]
    dist = -((zsq - 2.0 * prod) + esq)
    dist_ref[...] = dist
    m = jnp.max(dist, axis=1, keepdims=True)
    lane = lax.broadcasted_iota(jnp.int32, (BN, K), 1)
    ind = jnp.min(jnp.where(dist == m, lane, K), axis=1)
    ind_ref[0, 0, :] = ind
    onehot = (lane == ind[:, None]).astype(jnp.float32)
    q_ref[...] = lax.dot_general(onehot, e, (((1,), (0,)), ((), ())),
                                 preferred_element_type=jnp.float32)


@jax.jit
def kernel(x, embed):
    flat = x.reshape(N, DIM)
    dist, ind3, quant = pl.pallas_call(
        _vq_body,
        grid=(NB,),
        in_specs=[
            pl.BlockSpec((BN, DIM), lambda i: (i, 0)),
            pl.BlockSpec((K, DIM), lambda i: (0, 0)),
        ],
        out_specs=[
            pl.BlockSpec((BN, K), lambda i: (i, 0)),
            pl.BlockSpec((1, 1, BN), lambda i: (i, 0, 0)),
            pl.BlockSpec((BN, DIM), lambda i: (i, 0)),
        ],
        out_shape=[
            jax.ShapeDtypeStruct((N, K), jnp.float32),
            jax.ShapeDtypeStruct((NB, 1, BN), jnp.int32),
            jax.ShapeDtypeStruct((N, DIM), jnp.float32),
        ],
    )(flat, embed)
    embed_ind = ind3.reshape(B, T)
    quantize = quant.reshape(B, T, DIM)
    return quantize, embed_ind, dist
